# baseline (speedup 1.0000x reference)
import functools

import jax
import jax.numpy as jnp
from jax import lax
from jax.experimental import pallas as pl
from jax.experimental.pallas import tpu as pltpu
from jax.experimental.pallas import tpu_sc as plsc

_NUM_INSTANCE = 16384
_FEAT_LEN = 768
_POS_NUM = 8192
_NC, _NS = 2, 16
_NW = _NC * _NS
_ROWS_PER_W = _NUM_INSTANCE // _NW

_NBUF = 4
_CHUNK = 32
_NCHUNK = _ROWS_PER_W // _CHUNK

_TC_BLOCK = 1024
_TC_NBLK = _NUM_INSTANCE // _TC_BLOCK
_TC_FEAT_BLKS = _POS_NUM // _TC_BLOCK


def _sc_queue_update(feat, q):
    mesh = plsc.VectorSubcoreMesh(core_axis_name="c", subcore_axis_name="s")
    out_sds = jax.ShapeDtypeStruct((_NUM_INSTANCE, _FEAT_LEN), jnp.float32)

    @functools.partial(
        pl.kernel,
        out_type=out_sds,
        mesh=mesh,
        scratch_types=(
            [pltpu.VMEM((_CHUNK, _FEAT_LEN), jnp.float32)] * _NBUF
            + [pltpu.SemaphoreType.DMA] * (2 * _NBUF)
        ),
    )
    def body(feat_hbm, q_hbm, out_hbm, *scratch):
        bufs = scratch[:_NBUF]
        in_sems = scratch[_NBUF:2 * _NBUF]
        out_sems = scratch[2 * _NBUF:]
        wid = lax.axis_index("c") * _NS + lax.axis_index("s")
        base = wid * _ROWS_PER_W

        def run_pipeline(src):
            ins = [None] * _NCHUNK
            outs = [None] * _NCHUNK

            def start_in(c):
                ins[c] = pltpu.make_async_copy(
                    src.at[pl.ds(base + c * _CHUNK, _CHUNK)],
                    bufs[c % _NBUF], in_sems[c % _NBUF])
                ins[c].start()

            for c in range(_NBUF):
                start_in(c)
            for c in range(_NCHUNK):
                b = c % _NBUF
                ins[c].wait()
                outs[c] = pltpu.make_async_copy(
                    bufs[b], out_hbm.at[pl.ds(base + c * _CHUNK, _CHUNK)],
                    out_sems[b])
                outs[c].start()
                if c + _NBUF < _NCHUNK:
                    outs[c].wait()
                    start_in(c + _NBUF)
            for c in range(_NCHUNK - _NBUF, _NCHUNK):
                outs[c].wait()

        @pl.when(base < _POS_NUM)
        def _copy_feat():
            run_pipeline(feat_hbm)

        @pl.when(base >= _POS_NUM)
        def _copy_tail():
            run_pipeline(q_hbm)

    return body(feat, q)


def _tc_copy_body(feat_ref, q_ref, out_ref):
    i = pl.program_id(0)

    @pl.when(i < _TC_FEAT_BLKS)
    def _():
        out_ref[...] = feat_ref[...]

    @pl.when(i >= _TC_FEAT_BLKS)
    def _():
        out_ref[...] = q_ref[...]


def _tc_queue_update(feat, q):
    return pl.pallas_call(
        _tc_copy_body,
        grid=(_TC_NBLK,),
        in_specs=[
            pl.BlockSpec(
                (_TC_BLOCK, _FEAT_LEN),
                lambda i: (jnp.minimum(i, _TC_FEAT_BLKS - 1), 0)),
            pl.BlockSpec(
                (_TC_BLOCK, _FEAT_LEN),
                lambda i: (jnp.maximum(i, _TC_FEAT_BLKS), 0)),
        ],
        out_specs=pl.BlockSpec((_TC_BLOCK, _FEAT_LEN), lambda i: (i, 0)),
        out_shape=jax.ShapeDtypeStruct((_NUM_INSTANCE, _FEAT_LEN),
                                       jnp.float32),
    )(feat, q)


def kernel(vis_feat, lag_feat, vis_memory_queue, lag_memory_queue):
    new_vis = _sc_queue_update(vis_feat, vis_memory_queue)
    new_lag = _tc_queue_update(lag_feat, lag_memory_queue)
    return (new_vis, new_lag)


# speedup vs baseline: 1.0033x; 1.0033x over previous
import functools

import jax
import jax.numpy as jnp
from jax import lax
from jax.experimental import pallas as pl
from jax.experimental.pallas import tpu as pltpu
from jax.experimental.pallas import tpu_sc as plsc

_NUM_INSTANCE = 16384
_FEAT_LEN = 768
_POS_NUM = 8192
_NC, _NS = 2, 16
_NW = _NC * _NS
_ROWS_PER_W = _NUM_INSTANCE // _NW

_NBUF = 4
_CHUNK = 32
_NCHUNK = _ROWS_PER_W // _CHUNK

_TC_BLOCK = 1024
_TC_NBLK = _NUM_INSTANCE // _TC_BLOCK
_TC_FEAT_BLKS = _POS_NUM // _TC_BLOCK


def _sc_queue_update(feat, q):
    mesh = plsc.VectorSubcoreMesh(core_axis_name="c", subcore_axis_name="s")
    out_sds = jax.ShapeDtypeStruct((_NUM_INSTANCE, _FEAT_LEN), jnp.float32)

    @functools.partial(
        pl.kernel,
        out_type=out_sds,
        mesh=mesh,
        scratch_types=[
            pltpu.VMEM((_NBUF, _CHUNK, _FEAT_LEN), jnp.float32),
            pltpu.SemaphoreType.DMA((_NBUF,)),
            pltpu.SemaphoreType.DMA((_NBUF,)),
        ],
    )
    def body(feat_hbm, q_hbm, out_hbm, buf, in_sems, out_sems):
        wid = lax.axis_index("c") * _NS + lax.axis_index("s")
        base = wid * _ROWS_PER_W

        def run_pipeline(src):
            def in_copy(c):
                b = lax.rem(c, _NBUF)
                return pltpu.make_async_copy(
                    src.at[pl.ds(base + c * _CHUNK, _CHUNK)],
                    buf.at[b], in_sems.at[b])

            def out_copy(c):
                b = lax.rem(c, _NBUF)
                return pltpu.make_async_copy(
                    buf.at[b], out_hbm.at[pl.ds(base + c * _CHUNK, _CHUNK)],
                    out_sems.at[b])

            for c in range(_NBUF):
                in_copy(c).start()

            def step(c, carry):
                in_copy(c).wait()
                out_copy(c).start()

                @pl.when(c + _NBUF < _NCHUNK)
                def _():
                    out_copy(c).wait()
                    in_copy(c + _NBUF).start()

                return carry

            lax.fori_loop(0, _NCHUNK, step, 0)
            for c in range(_NCHUNK - _NBUF, _NCHUNK):
                out_copy(c).wait()

        @pl.when(base < _POS_NUM)
        def _copy_feat():
            run_pipeline(feat_hbm)

        @pl.when(base >= _POS_NUM)
        def _copy_tail():
            run_pipeline(q_hbm)

    return body(feat, q)


def _tc_copy_body(feat_ref, q_ref, out_ref):
    i = pl.program_id(0)

    @pl.when(i < _TC_FEAT_BLKS)
    def _():
        out_ref[...] = feat_ref[...]

    @pl.when(i >= _TC_FEAT_BLKS)
    def _():
        out_ref[...] = q_ref[...]


def _tc_queue_update(feat, q):
    return pl.pallas_call(
        _tc_copy_body,
        grid=(_TC_NBLK,),
        in_specs=[
            pl.BlockSpec(
                (_TC_BLOCK, _FEAT_LEN),
                lambda i: (jnp.minimum(i, _TC_FEAT_BLKS - 1), 0)),
            pl.BlockSpec(
                (_TC_BLOCK, _FEAT_LEN),
                lambda i: (jnp.maximum(i, _TC_FEAT_BLKS), 0)),
        ],
        out_specs=pl.BlockSpec((_TC_BLOCK, _FEAT_LEN), lambda i: (i, 0)),
        out_shape=jax.ShapeDtypeStruct((_NUM_INSTANCE, _FEAT_LEN),
                                       jnp.float32),
    )(feat, q)


def kernel(vis_feat, lag_feat, vis_memory_queue, lag_memory_queue):
    new_vis = _sc_queue_update(vis_feat, vis_memory_queue)
    new_lag = _tc_queue_update(lag_feat, lag_memory_queue)
    return (new_vis, new_lag)


# speedup vs baseline: 1.0670x; 1.0634x over previous
import functools

import jax
import jax.numpy as jnp
from jax import lax
from jax.experimental import pallas as pl
from jax.experimental.pallas import tpu as pltpu
from jax.experimental.pallas import tpu_sc as plsc

_NUM_INSTANCE = 16384
_FEAT_LEN = 768
_POS_NUM = 8192

_SP_NBUF = 4
_SP_LEAD = 2
_SP_CHUNK = 512
_SP_NCHUNK = _POS_NUM // _SP_CHUNK

_TC_BLOCK = 1024
_TC_NBLK = _NUM_INSTANCE // _TC_BLOCK
_TC_FEAT_BLKS = _POS_NUM // _TC_BLOCK


def _sc_queue_update(feat, q):
    mesh = plsc.ScalarSubcoreMesh(axis_name="c")
    out_sds = jax.ShapeDtypeStruct((_NUM_INSTANCE, _FEAT_LEN), jnp.float32)

    @functools.partial(
        pl.kernel,
        out_type=out_sds,
        mesh=mesh,
        scratch_types=[
            pltpu.VMEM_SHARED((_SP_NBUF, _SP_CHUNK, _FEAT_LEN), jnp.float32),
            pltpu.SemaphoreType.DMA((_SP_NBUF,)),
            pltpu.SemaphoreType.DMA((_SP_NBUF,)),
        ],
    )
    def body(feat_hbm, q_hbm, out_hbm, buf, in_sems, out_sems):
        core = lax.axis_index("c")
        base = core * _POS_NUM

        def run_sp_pipeline(src):
            def in_copy(c):
                b = lax.rem(c, _SP_NBUF)
                return pltpu.make_async_copy(
                    src.at[pl.ds(base + c * _SP_CHUNK, _SP_CHUNK)],
                    buf.at[b], in_sems.at[b])

            def out_copy(c):
                b = lax.rem(c, _SP_NBUF)
                return pltpu.make_async_copy(
                    buf.at[b],
                    out_hbm.at[pl.ds(base + c * _SP_CHUNK, _SP_CHUNK)],
                    out_sems.at[b])

            for c in range(_SP_LEAD):
                in_copy(c).start()

            def step(c, carry):
                in_copy(c).wait()
                out_copy(c).start()
                j = c + _SP_LEAD

                @pl.when(j < _SP_NCHUNK)
                def _():
                    @pl.when(j >= _SP_NBUF)
                    def _():
                        out_copy(j - _SP_NBUF).wait()

                    in_copy(j).start()

                return carry

            lax.fori_loop(0, _SP_NCHUNK, step, 0)
            for c in range(_SP_NCHUNK - _SP_NBUF, _SP_NCHUNK):
                out_copy(c).wait()

        @pl.when(core == 0)
        def _():
            run_sp_pipeline(feat_hbm)

        @pl.when(core == 1)
        def _():
            run_sp_pipeline(q_hbm)

    return body(feat, q)


def _tc_copy_body(feat_ref, q_ref, out_ref):
    i = pl.program_id(0)

    @pl.when(i < _TC_FEAT_BLKS)
    def _():
        out_ref[...] = feat_ref[...]

    @pl.when(i >= _TC_FEAT_BLKS)
    def _():
        out_ref[...] = q_ref[...]


def _tc_queue_update(feat, q):
    return pl.pallas_call(
        _tc_copy_body,
        grid=(_TC_NBLK,),
        in_specs=[
            pl.BlockSpec(
                (_TC_BLOCK, _FEAT_LEN),
                lambda i: (jnp.minimum(i, _TC_FEAT_BLKS - 1), 0)),
            pl.BlockSpec(
                (_TC_BLOCK, _FEAT_LEN),
                lambda i: (jnp.maximum(i, _TC_FEAT_BLKS), 0)),
        ],
        out_specs=pl.BlockSpec((_TC_BLOCK, _FEAT_LEN), lambda i: (i, 0)),
        out_shape=jax.ShapeDtypeStruct((_NUM_INSTANCE, _FEAT_LEN),
                                       jnp.float32),
    )(feat, q)


def kernel(vis_feat, lag_feat, vis_memory_queue, lag_memory_queue):
    new_lag = _tc_queue_update(lag_feat, lag_memory_queue)
    new_vis = _sc_queue_update(vis_feat, vis_memory_queue)
    return (new_vis, new_lag)
